# baseline (speedup 1.0000x reference)
import functools

import jax
import jax.numpy as jnp
from jax import lax
from jax.experimental import pallas as pl
from jax.experimental.pallas import tpu as pltpu
from jax.experimental.pallas import tpu_sc as plsc

GS = 8
CELL = 64
WGS = 3
WS = GS - WGS + 1
TOP_K = 4
BB = 8


def _fused_kernel(x_ref, out_ref):
    f32 = jnp.float32
    rows = BB * GS
    t = x_ref[...].reshape(rows, CELL, 512)
    y = jnp.sum(t, axis=1)
    c_i = lax.broadcasted_iota(jnp.int32, (512, 64), 0) // 8
    m_i = lax.broadcasted_iota(jnp.int32, (512, 64), 1)
    pa = (c_i == m_i).astype(f32)
    z1 = lax.dot_general(y, pa, (((1,), (0,)), ((), ())),
                         precision=lax.Precision.HIGHEST,
                         preferred_element_type=f32)
    d_i = lax.broadcasted_iota(jnp.int32, (64, GS), 0) // 8
    j_i = lax.broadcasted_iota(jnp.int32, (64, GS), 1)
    pb = (d_i == j_i).astype(f32)
    z2 = lax.dot_general(z1, pb, (((1,), (0,)), ((), ())),
                         precision=lax.Precision.HIGHEST,
                         preferred_element_type=f32)
    z2 = z2 * (1.0 / (CELL * CELL))
    z3 = z2.reshape(BB, GS, GS)
    g = jnp.concatenate([z3[:, gi, :] for gi in range(GS)], axis=1)

    _window_topk(g, out_ref)


def _window_topk(g, out_ref):
    f32 = jnp.float32
    b = g.shape[0]
    nl = GS * WS
    g = jnp.concatenate([g, jnp.zeros((b, 2 * GS), f32)], axis=1)
    w = jnp.zeros((b, nl), f32)
    for di in range(WGS):
        for dj in range(WGS):
            o = GS * di + dj
            w = w + g[:, o:o + nl]
    lane = lax.broadcasted_iota(jnp.int32, (b, nl), 1)
    wi = lane // GS
    wj = lane % GS
    idx = WS * wi + wj
    neg = jnp.float32(-jnp.inf)
    big = jnp.int32(WS * WS)
    w = jnp.where(wj < WS, w, neg)
    lane8 = lax.broadcasted_iota(jnp.int32, (b, 2 * TOP_K), 1)
    out = jnp.zeros((b, 2 * TOP_K), jnp.int32)
    for k in range(TOP_K):
        m = jnp.max(w, axis=1, keepdims=True)
        cand = jnp.where(w == m, idx, big)
        amin = jnp.min(cand, axis=1, keepdims=True)
        w = jnp.where(idx == amin, neg, w)
        row = amin // WS
        col = amin % WS
        out = jnp.where(lane8 == 2 * k, row, out)
        out = jnp.where(lane8 == 2 * k + 1, col, out)
    out_ref[...] = out


CH = CELL * 512
NW = 32
L = 16


def _sc_reduce():
    npix = 512 * 512
    mesh = plsc.VectorSubcoreMesh(core_axis_name="c", subcore_axis_name="s")

    @functools.partial(
        pl.kernel, mesh=mesh,
        out_type=jax.ShapeDtypeStruct((NW * 64 * L,), jnp.float32),
        scratch_types=[
            pltpu.VMEM((2, CH), jnp.float32),
            pltpu.VMEM((GS * 512,), jnp.float32),
            pltpu.VMEM((2 * 64 * L + L,), jnp.float32),
            pltpu.VMEM((64 * L + L,), jnp.float32),
            pltpu.VMEM((64 * L + L,), jnp.float32),
            pltpu.SemaphoreType.DMA((2,)),
        ],
    )
    def sc_kernel(x_hbm, out_hbm, buf, colsum, fb1, fb2, fb3, sem):
        f32 = jnp.float32
        wid = lax.axis_index("s") * 2 + lax.axis_index("c")
        zero16 = jnp.zeros((L,), f32)

        def start(gi):
            off = wid * npix + gi * CH
            return pltpu.async_copy(
                x_hbm.at[pl.ds(off, CH)], buf.at[gi % 2], sem.at[gi % 2])

        cp = start(0)
        for gi in range(GS):
            cp.wait()
            if gi + 1 < GS:
                cp = start(gi + 1)

            def body(r2, accs):
                base = r2 * 1024
                out = []
                for c in range(32):
                    a = buf[gi % 2, pl.ds(base + c * L, L)]
                    a2 = buf[gi % 2, pl.ds(base + 512 + c * L, L)]
                    out.append(accs[c] + (a + a2))
                return tuple(out)

            accs = lax.fori_loop(0, 32, body, tuple([zero16] * 32))
            for c in range(32):
                colsum[pl.ds(gi * 512 + c * L, L)] = accs[c]

        for c in range(64):
            fb1[pl.ds(32 * c, L)] = (colsum[pl.ds(64 * c, L)]
                                     + colsum[pl.ds(64 * c + 32, L)])
            fb1[pl.ds(32 * c + L, L)] = (colsum[pl.ds(64 * c + L, L)]
                                         + colsum[pl.ds(64 * c + 48, L)])
        for c in range(64):
            fb2[pl.ds(L * c, L)] = (fb1[pl.ds(32 * c, L)]
                                    + fb1[pl.ds(32 * c + L, L)])
        for src_ref, dst_ref, wdt in ((fb2, fb3, 8), (fb3, fb2, 4),
                                      (fb2, fb3, 2), (fb3, fb2, 1)):
            for c in range(64):
                dst_ref[pl.ds(L * c, L)] = (src_ref[pl.ds(L * c, L)]
                                            + src_ref[pl.ds(L * c + wdt, L)])
        pltpu.sync_copy(fb2.at[pl.ds(0, 64 * L)],
                        out_hbm.at[pl.ds(wid * 64 * L, 64 * L)])

    return sc_kernel


def _finish_kernel(g_ref, out_ref):
    g = g_ref[...] * (1.0 / (CELL * CELL))
    _window_topk(g, out_ref)


def _tc_coords(x, nbatch):
    nsteps = nbatch // BB
    return pl.pallas_call(
        _fused_kernel,
        grid=(nsteps,),
        in_specs=[pl.BlockSpec((BB * 512, 512), lambda i: (i, 0))],
        out_specs=pl.BlockSpec((BB, 2 * TOP_K), lambda i: (i, 0)),
        out_shape=jax.ShapeDtypeStruct((nbatch, 2 * TOP_K), jnp.int32),
    )(x)


def kernel(sampling_map):
    b, c, h, w = sampling_map.shape
    half = b // 2
    x = sampling_map.reshape(b * h, w)
    coords_tc = _tc_coords(x[:half * h], half)
    folded = _sc_reduce()(sampling_map.reshape(b * h * w)[half * h * w:])
    gsums = folded.reshape(half * 64, L)[:, 0].reshape(half, 64)
    coords_sc = pl.pallas_call(
        _finish_kernel,
        out_shape=jax.ShapeDtypeStruct((half, 2 * TOP_K), jnp.int32),
    )(gsums)
    coords = jnp.concatenate([coords_tc, coords_sc], axis=0)
    return coords.reshape(b, TOP_K, 2)


# speedup vs baseline: 1.1702x; 1.1702x over previous
import functools

import jax
import jax.numpy as jnp
from jax import lax
from jax.experimental import pallas as pl
from jax.experimental.pallas import tpu as pltpu
from jax.experimental.pallas import tpu_sc as plsc

GS = 8
CELL = 64
WGS = 3
WS = GS - WGS + 1
TOP_K = 4
BB = 8


def _fused_kernel(x_ref, out_ref):
    f32 = jnp.float32
    rows = BB * GS
    t = x_ref[...].reshape(rows, CELL, 512)
    y = jnp.sum(t, axis=1)
    c_i = lax.broadcasted_iota(jnp.int32, (512, 64), 0) // 8
    m_i = lax.broadcasted_iota(jnp.int32, (512, 64), 1)
    pa = (c_i == m_i).astype(f32)
    z1 = lax.dot_general(y, pa, (((1,), (0,)), ((), ())),
                         precision=lax.Precision.HIGHEST,
                         preferred_element_type=f32)
    d_i = lax.broadcasted_iota(jnp.int32, (64, GS), 0) // 8
    j_i = lax.broadcasted_iota(jnp.int32, (64, GS), 1)
    pb = (d_i == j_i).astype(f32)
    z2 = lax.dot_general(z1, pb, (((1,), (0,)), ((), ())),
                         precision=lax.Precision.HIGHEST,
                         preferred_element_type=f32)
    z2 = z2 * (1.0 / (CELL * CELL))
    z3 = z2.reshape(BB, GS, GS)
    g = jnp.concatenate([z3[:, gi, :] for gi in range(GS)], axis=1)

    _window_topk(g, out_ref)


def _window_topk(g, out_ref):
    f32 = jnp.float32
    b = g.shape[0]
    nl = GS * WS
    g = jnp.concatenate([g, jnp.zeros((b, 2 * GS), f32)], axis=1)
    w = jnp.zeros((b, nl), f32)
    for di in range(WGS):
        for dj in range(WGS):
            o = GS * di + dj
            w = w + g[:, o:o + nl]
    lane = lax.broadcasted_iota(jnp.int32, (b, nl), 1)
    wi = lane // GS
    wj = lane % GS
    idx = WS * wi + wj
    neg = jnp.float32(-jnp.inf)
    big = jnp.int32(WS * WS)
    w = jnp.where(wj < WS, w, neg)
    lane8 = lax.broadcasted_iota(jnp.int32, (b, 2 * TOP_K), 1)
    out = jnp.zeros((b, 2 * TOP_K), jnp.int32)
    for k in range(TOP_K):
        m = jnp.max(w, axis=1, keepdims=True)
        cand = jnp.where(w == m, idx, big)
        amin = jnp.min(cand, axis=1, keepdims=True)
        w = jnp.where(idx == amin, neg, w)
        row = amin // WS
        col = amin % WS
        out = jnp.where(lane8 == 2 * k, row, out)
        out = jnp.where(lane8 == 2 * k + 1, col, out)
    out_ref[...] = out


CH = CELL * 512
NW = 32
L = 16
SC_NB = 32
SC_BASE = SC_NB * 512 * 512


def _sc_reduce():
    npix = 512 * 512
    mesh = plsc.VectorSubcoreMesh(core_axis_name="c", subcore_axis_name="s")

    @functools.partial(
        pl.kernel, mesh=mesh,
        out_type=jax.ShapeDtypeStruct((NW * 64 * L,), jnp.float32),
        scratch_types=[
            pltpu.VMEM((2, CH), jnp.float32),
            pltpu.VMEM((GS * 512,), jnp.float32),
            pltpu.VMEM((2 * 64 * L + L,), jnp.float32),
            pltpu.VMEM((64 * L + L,), jnp.float32),
            pltpu.VMEM((64 * L + L,), jnp.float32),
            pltpu.SemaphoreType.DMA((2,)),
        ],
    )
    def sc_kernel(x_hbm, out_hbm, buf, colsum, fb1, fb2, fb3, sem):
        f32 = jnp.float32
        wid = lax.axis_index("s") * 2 + lax.axis_index("c")
        zero16 = jnp.zeros((L,), f32)

        def start(gi):
            off = SC_BASE + wid * npix + gi * CH
            return pltpu.async_copy(
                x_hbm.at[pl.ds(off, CH)], buf.at[gi % 2], sem.at[gi % 2])

        cp = start(0)
        for gi in range(GS):
            cp.wait()
            if gi + 1 < GS:
                cp = start(gi + 1)

            def body(r4, accs):
                base = r4 * (4 * 512)
                out = list(accs)
                for c in range(32):
                    s = None
                    for r in range(4):
                        a = buf[gi % 2, pl.ds(base + r * 512 + c * L, L)]
                        s = a if s is None else s + a
                    out[c] = out[c] + s
                return tuple(out)

            accs = lax.fori_loop(0, 16, body, tuple([zero16] * 32))
            for c in range(32):
                colsum[pl.ds(gi * 512 + c * L, L)] = accs[c]

        for c in range(64):
            fb1[pl.ds(32 * c, L)] = (colsum[pl.ds(64 * c, L)]
                                     + colsum[pl.ds(64 * c + 32, L)])
            fb1[pl.ds(32 * c + L, L)] = (colsum[pl.ds(64 * c + L, L)]
                                         + colsum[pl.ds(64 * c + 48, L)])
        for c in range(64):
            fb2[pl.ds(L * c, L)] = (fb1[pl.ds(32 * c, L)]
                                    + fb1[pl.ds(32 * c + L, L)])
        for src_ref, dst_ref, wdt in ((fb2, fb3, 8), (fb3, fb2, 4),
                                      (fb2, fb3, 2), (fb3, fb2, 1)):
            for c in range(64):
                dst_ref[pl.ds(L * c, L)] = (src_ref[pl.ds(L * c, L)]
                                            + src_ref[pl.ds(L * c + wdt, L)])
        pltpu.sync_copy(fb2.at[pl.ds(0, 64 * L)],
                        out_hbm.at[pl.ds(wid * 64 * L, 64 * L)])

    return sc_kernel


def _finish_kernel(g_ref, out_ref):
    g = g_ref[...] * (1.0 / (CELL * CELL))
    _window_topk(g, out_ref)


def _tc_coords(x, nbatch):
    nsteps = nbatch // BB
    return pl.pallas_call(
        _fused_kernel,
        grid=(nsteps,),
        in_specs=[pl.BlockSpec((BB * 512, 512), lambda i: (i, 0))],
        out_specs=pl.BlockSpec((BB, 2 * TOP_K), lambda i: (i, 0)),
        out_shape=jax.ShapeDtypeStruct((nbatch, 2 * TOP_K), jnp.int32),
    )(x)


def kernel(sampling_map):
    b, c, h, w = sampling_map.shape
    half = b // 2
    x = sampling_map.reshape(b * h, w)
    coords_tc = _tc_coords(x[:half * h], half)
    folded = _sc_reduce()(sampling_map.reshape(b * h * w))
    gsums = folded.reshape(half * 64, L)[:, 0].reshape(half, 64)
    coords_sc = pl.pallas_call(
        _finish_kernel,
        out_shape=jax.ShapeDtypeStruct((half, 2 * TOP_K), jnp.int32),
    )(gsums)
    coords = jnp.concatenate([coords_tc, coords_sc], axis=0)
    return coords.reshape(b, TOP_K, 2)


# speedup vs baseline: 2.0472x; 1.7494x over previous
import functools

import jax
import jax.numpy as jnp
from jax import lax
from jax.experimental import pallas as pl
from jax.experimental.pallas import tpu as pltpu
from jax.experimental.pallas import tpu_sc as plsc

GS = 8
CELL = 64
WGS = 3
WS = GS - WGS + 1
TOP_K = 4
BB = 8


def _fused_kernel(x_ref, out_ref):
    f32 = jnp.float32
    rows = BB * GS
    t = x_ref[...].reshape(rows, CELL, 512)
    y = jnp.sum(t, axis=1)
    c_i = lax.broadcasted_iota(jnp.int32, (512, 64), 0) // 8
    m_i = lax.broadcasted_iota(jnp.int32, (512, 64), 1)
    pa = (c_i == m_i).astype(f32)
    z1 = lax.dot_general(y, pa, (((1,), (0,)), ((), ())),
                         precision=lax.Precision.HIGHEST,
                         preferred_element_type=f32)
    d_i = lax.broadcasted_iota(jnp.int32, (64, GS), 0) // 8
    j_i = lax.broadcasted_iota(jnp.int32, (64, GS), 1)
    pb = (d_i == j_i).astype(f32)
    z2 = lax.dot_general(z1, pb, (((1,), (0,)), ((), ())),
                         precision=lax.Precision.HIGHEST,
                         preferred_element_type=f32)
    z2 = z2 * (1.0 / (CELL * CELL))
    z3 = z2.reshape(BB, GS, GS)
    g = jnp.concatenate([z3[:, gi, :] for gi in range(GS)], axis=1)

    _window_topk(g, out_ref)


def _window_topk(g, out_ref):
    f32 = jnp.float32
    b = g.shape[0]
    nl = GS * WS
    g = jnp.concatenate([g, jnp.zeros((b, 2 * GS), f32)], axis=1)
    w = jnp.zeros((b, nl), f32)
    for di in range(WGS):
        for dj in range(WGS):
            o = GS * di + dj
            w = w + g[:, o:o + nl]
    lane = lax.broadcasted_iota(jnp.int32, (b, nl), 1)
    wi = lane // GS
    wj = lane % GS
    idx = WS * wi + wj
    neg = jnp.float32(-jnp.inf)
    big = jnp.int32(WS * WS)
    w = jnp.where(wj < WS, w, neg)
    lane8 = lax.broadcasted_iota(jnp.int32, (b, 2 * TOP_K), 1)
    out = jnp.zeros((b, 2 * TOP_K), jnp.int32)
    for k in range(TOP_K):
        m = jnp.max(w, axis=1, keepdims=True)
        cand = jnp.where(w == m, idx, big)
        amin = jnp.min(cand, axis=1, keepdims=True)
        w = jnp.where(idx == amin, neg, w)
        row = amin // WS
        col = amin % WS
        out = jnp.where(lane8 == 2 * k, row, out)
        out = jnp.where(lane8 == 2 * k + 1, col, out)
    out_ref[...] = out


CH = CELL * 512
NW = 32
L = 16
SC_NB = 32
SC_BASE = SC_NB * 512 * 512


def _sc_reduce():
    npix = 512 * 512
    mesh = plsc.VectorSubcoreMesh(core_axis_name="c", subcore_axis_name="s")

    @functools.partial(
        pl.kernel, mesh=mesh,
        out_type=jax.ShapeDtypeStruct((NW * 64 * L,), jnp.float32),
        scratch_types=[
            pltpu.VMEM((2, CELL, 512), jnp.float32),
            pltpu.VMEM((GS * 512,), jnp.float32),
            pltpu.VMEM((2 * 64 * L + L,), jnp.float32),
            pltpu.VMEM((64 * L + L,), jnp.float32),
            pltpu.VMEM((64 * L + L,), jnp.float32),
            pltpu.SemaphoreType.DMA((2,)),
        ],
    )
    def sc_kernel(x_hbm, out_hbm, buf, colsum, fb1, fb2, fb3, sem):
        f32 = jnp.float32
        wid = lax.axis_index("s") * 2 + lax.axis_index("c")
        zero16 = jnp.zeros((L,), f32)

        def start(gi):
            row0 = (SC_NB + wid) * 512 + gi * CELL
            return pltpu.async_copy(
                x_hbm.at[pl.ds(row0, CELL), :], buf.at[gi % 2],
                sem.at[gi % 2])

        cp = start(0)
        for gi in range(GS):
            cp.wait()
            if gi + 1 < GS:
                cp = start(gi + 1)

            def body(r4, accs):
                out = list(accs)
                for c in range(32):
                    s = None
                    for r in range(4):
                        a = buf[gi % 2, r4 * 4 + r, pl.ds(c * L, L)]
                        s = a if s is None else s + a
                    out[c] = out[c] + s
                return tuple(out)

            accs = lax.fori_loop(0, 16, body, tuple([zero16] * 32))
            for c in range(32):
                colsum[pl.ds(gi * 512 + c * L, L)] = accs[c]

        for c in range(64):
            fb1[pl.ds(32 * c, L)] = (colsum[pl.ds(64 * c, L)]
                                     + colsum[pl.ds(64 * c + 32, L)])
            fb1[pl.ds(32 * c + L, L)] = (colsum[pl.ds(64 * c + L, L)]
                                         + colsum[pl.ds(64 * c + 48, L)])
        for c in range(64):
            fb2[pl.ds(L * c, L)] = (fb1[pl.ds(32 * c, L)]
                                    + fb1[pl.ds(32 * c + L, L)])
        for src_ref, dst_ref, wdt in ((fb2, fb3, 8), (fb3, fb2, 4),
                                      (fb2, fb3, 2), (fb3, fb2, 1)):
            for c in range(64):
                dst_ref[pl.ds(L * c, L)] = (src_ref[pl.ds(L * c, L)]
                                            + src_ref[pl.ds(L * c + wdt, L)])
        pltpu.sync_copy(fb2.at[pl.ds(0, 64 * L)],
                        out_hbm.at[pl.ds(wid * 64 * L, 64 * L)])

    return sc_kernel


def _finish_kernel(g_ref, out_ref):
    g = g_ref[...] * (1.0 / (CELL * CELL))
    _window_topk(g, out_ref)


def _tc_coords(x, nbatch):
    nsteps = nbatch // BB
    return pl.pallas_call(
        _fused_kernel,
        grid=(nsteps,),
        in_specs=[pl.BlockSpec((BB * 512, 512), lambda i: (i, 0))],
        out_specs=pl.BlockSpec((BB, 2 * TOP_K), lambda i: (i, 0)),
        out_shape=jax.ShapeDtypeStruct((nbatch, 2 * TOP_K), jnp.int32),
    )(x)


def kernel(sampling_map):
    b, c, h, w = sampling_map.shape
    half = b // 2
    x = sampling_map.reshape(b * h, w)
    coords_tc = _tc_coords(x[:half * h], half)
    folded = _sc_reduce()(x)
    gsums = folded.reshape(half * 64, L)[:, 0].reshape(half, 64)
    coords_sc = pl.pallas_call(
        _finish_kernel,
        out_shape=jax.ShapeDtypeStruct((half, 2 * TOP_K), jnp.int32),
    )(gsums)
    coords = jnp.concatenate([coords_tc, coords_sc], axis=0)
    return coords.reshape(b, TOP_K, 2)


# speedup vs baseline: 4.8634x; 2.3757x over previous
import jax
import jax.numpy as jnp
from jax import lax
from jax.experimental import pallas as pl

GS = 8
CELL = 64
WGS = 3
WS = GS - WGS + 1
TOP_K = 4
BB = 8


def _fused_kernel(x_ref, out_ref):
    f32 = jnp.float32
    rows = BB * GS
    t = x_ref[...].reshape(rows, CELL, 512)
    y = jnp.sum(t, axis=1)
    c_i = lax.broadcasted_iota(jnp.int32, (512, 64), 0) // 8
    m_i = lax.broadcasted_iota(jnp.int32, (512, 64), 1)
    pa = (c_i == m_i).astype(f32)
    z1 = lax.dot_general(y, pa, (((1,), (0,)), ((), ())),
                         precision=lax.Precision.HIGHEST,
                         preferred_element_type=f32)
    d_i = lax.broadcasted_iota(jnp.int32, (64, GS), 0) // 8
    j_i = lax.broadcasted_iota(jnp.int32, (64, GS), 1)
    pb = (d_i == j_i).astype(f32)
    z2 = lax.dot_general(z1, pb, (((1,), (0,)), ((), ())),
                         precision=lax.Precision.HIGHEST,
                         preferred_element_type=f32)
    z2 = z2 * (1.0 / (CELL * CELL))
    z3 = z2.reshape(BB, GS, GS)
    g = jnp.concatenate([z3[:, gi, :] for gi in range(GS)], axis=1)

    _window_topk(g, out_ref)


def _window_topk(g, out_ref):
    f32 = jnp.float32
    b = g.shape[0]
    nl = GS * WS
    g = jnp.concatenate([g, jnp.zeros((b, 2 * GS), f32)], axis=1)
    w = jnp.zeros((b, nl), f32)
    for di in range(WGS):
        for dj in range(WGS):
            o = GS * di + dj
            w = w + g[:, o:o + nl]
    lane = lax.broadcasted_iota(jnp.int32, (b, nl), 1)
    wi = lane // GS
    wj = lane % GS
    idx = WS * wi + wj
    neg = jnp.float32(-jnp.inf)
    big = jnp.int32(WS * WS)
    w = jnp.where(wj < WS, w, neg)
    lane8 = lax.broadcasted_iota(jnp.int32, (b, 2 * TOP_K), 1)
    out = jnp.zeros((b, 2 * TOP_K), jnp.int32)
    for k in range(TOP_K):
        m = jnp.max(w, axis=1, keepdims=True)
        cand = jnp.where(w == m, idx, big)
        amin = jnp.min(cand, axis=1, keepdims=True)
        w = jnp.where(idx == amin, neg, w)
        row = amin // WS
        col = amin % WS
        out = jnp.where(lane8 == 2 * k, row, out)
        out = jnp.where(lane8 == 2 * k + 1, col, out)
    out_ref[...] = out


def _tc_coords(x, nbatch):
    nsteps = nbatch // BB
    return pl.pallas_call(
        _fused_kernel,
        grid=(nsteps,),
        in_specs=[pl.BlockSpec((BB * 512, 512), lambda i: (i, 0))],
        out_specs=pl.BlockSpec((BB, 2 * TOP_K), lambda i: (i, 0)),
        out_shape=jax.ShapeDtypeStruct((nbatch, 2 * TOP_K), jnp.int32),
    )(x)


def kernel(sampling_map):
    b, c, h, w = sampling_map.shape
    coords = _tc_coords(sampling_map.reshape(b * h, w), b)
    return coords.reshape(b, TOP_K, 2)
